# baseline (speedup 1.0000x reference)
import functools

import jax
import jax.numpy as jnp
from jax import lax
from jax.experimental import pallas as pl
from jax.experimental.pallas import tpu as pltpu
from jax.experimental.pallas import tpu_sc as plsc

B = 16384
L = 20
R = 8
PAD = 100000
TBL = PAD + 1

NC = 2
NS = 16
NW = NC * NS
RW = B // NW
NE = RW * L
CHUNK = 128
NCHUNK = NE // CHUNK
VPC = CHUNK // 16


def _body(m_hbm, p_hbm, r_hbm, node_hbm, signs_hbm,
          bg_hbm, meta_hbm, pers_hbm, out_hbm,
          m_v, p_v, r_v, node_v, signs_v,
          idx_bg, idx_meta, idx_pers,
          val_bg, val_meta, val_pers,
          lp_v, out_v, sem_bg, sem_meta, sem_pers):
    wid = lax.axis_index("s") * NC + lax.axis_index("c")
    base_row = wid * RW
    base_el = wid * NE

    pltpu.sync_copy(m_hbm.at[pl.ds(base_row, RW)], m_v)
    pltpu.sync_copy(p_hbm.at[pl.ds(base_row, RW)], p_v)
    pltpu.sync_copy(r_hbm.at[pl.ds(base_row, RW)], r_v)
    pltpu.sync_copy(node_hbm.at[pl.ds(base_el, NE)], node_v)
    pltpu.sync_copy(signs_hbm.at[pl.ds(base_el, NE)], signs_v)

    iota16 = lax.iota(jnp.int32, 16)

    def fire(j, carry):
        for cc in range(VPC):
            q0 = j * CHUNK + cc * 16
            q = iota16 + q0
            b = ((q.astype(jnp.float32) + 0.5) * (1.0 / L)).astype(jnp.int32)
            rv = plsc.load_gather(r_v, [b])
            mv = plsc.load_gather(m_v, [b])
            pv = plsc.load_gather(p_v, [b])
            nd = node_v[pl.ds(q0, 16)]
            t = rv * TBL + nd
            sl = pl.ds(cc * 16, 16)
            idx_bg[j, sl] = t
            idx_meta[j, sl] = mv * (R * TBL) + t
            idx_pers[j, sl] = pv * (R * TBL) + t
        pltpu.async_copy(bg_hbm.at[idx_bg.at[j]], val_bg.at[j], sem_bg)
        pltpu.async_copy(meta_hbm.at[idx_meta.at[j]], val_meta.at[j], sem_meta)
        pltpu.async_copy(pers_hbm.at[idx_pers.at[j]], val_pers.at[j], sem_pers)
        return carry

    lax.fori_loop(0, NCHUNK, fire, 0)

    def drain(j, carry):
        pltpu.make_async_copy(bg_hbm.at[idx_bg.at[j]], val_bg.at[j], sem_bg).wait()
        pltpu.make_async_copy(meta_hbm.at[idx_meta.at[j]], val_meta.at[j], sem_meta).wait()
        pltpu.make_async_copy(pers_hbm.at[idx_pers.at[j]], val_pers.at[j], sem_pers).wait()
        return carry

    lax.fori_loop(0, NCHUNK, drain, 0)

    def logsig(j, carry):
        for cc in range(VPC):
            q0 = j * CHUNK + cc * 16
            sl = pl.ds(cc * 16, 16)
            logits = val_bg[j, sl] + val_meta[j, sl] + val_pers[j, sl]
            x = signs_v[pl.ds(q0, 16)] * logits
            u = jnp.exp(-jnp.abs(x))
            s = u / (2.0 + u)
            s2 = s * s
            l1p = 2.0 * s * (1.0 + s2 * (1.0 / 3.0 + s2 * (0.2 + s2 * (1.0 / 7.0 + s2 * (1.0 / 9.0)))))
            lp = jnp.minimum(x, 0.0) - l1p
            nd = node_v[pl.ds(q0, 16)]
            lp_v[pl.ds(q0, 16)] = jnp.where(nd == PAD, 0.0, lp)
        return carry

    lax.fori_loop(0, NCHUNK, logsig, 0)

    def rowsum(c, carry):
        fb = (iota16 + c * 16) * L
        acc = plsc.load_gather(lp_v, [fb])
        for l in range(1, L):
            acc = acc + plsc.load_gather(lp_v, [fb + l])
        out_v[pl.ds(c * 16, 16)] = acc
        return carry

    lax.fori_loop(0, RW // 16, rowsum, 0)

    pltpu.sync_copy(out_v, out_hbm.at[pl.ds(base_row, RW)])


@jax.jit
def _sage_sc(m, p, r, node, signs, bg, meta, pers):
    mesh = plsc.VectorSubcoreMesh(
        core_axis_name="c", subcore_axis_name="s",
        num_cores=NC, num_subcores=NS)
    f = pl.kernel(
        _body,
        out_type=jax.ShapeDtypeStruct((B,), jnp.float32),
        mesh=mesh,
        compiler_params=pltpu.CompilerParams(needs_layout_passes=False),
        scratch_types=[
            pltpu.VMEM((RW,), jnp.int32),
            pltpu.VMEM((RW,), jnp.int32),
            pltpu.VMEM((RW,), jnp.int32),
            pltpu.VMEM((NE,), jnp.int32),
            pltpu.VMEM((NE,), jnp.float32),
            pltpu.VMEM((NCHUNK, CHUNK), jnp.int32),
            pltpu.VMEM((NCHUNK, CHUNK), jnp.int32),
            pltpu.VMEM((NCHUNK, CHUNK), jnp.int32),
            pltpu.VMEM((NCHUNK, CHUNK), jnp.float32),
            pltpu.VMEM((NCHUNK, CHUNK), jnp.float32),
            pltpu.VMEM((NCHUNK, CHUNK), jnp.float32),
            pltpu.VMEM((NE,), jnp.float32),
            pltpu.VMEM((RW,), jnp.float32),
            pltpu.SemaphoreType.DMA,
            pltpu.SemaphoreType.DMA,
            pltpu.SemaphoreType.DMA,
        ],
    )
    return f(m, p, r, node, signs, bg, meta, pers)


def kernel(m_idx, p_idx, r_idx, node_paths, node_signs,
           eta_bg, eta_meta, eta_pers):
    m = m_idx.astype(jnp.int32)
    p = p_idx.astype(jnp.int32)
    r = r_idx.astype(jnp.int32)
    node = node_paths.astype(jnp.int32).reshape(-1)
    signs = node_signs.reshape(-1)
    bg = eta_bg.reshape(-1)
    meta = eta_meta.reshape(-1)
    pers = eta_pers.reshape(-1)
    return _sage_sc(m, p, r, node, signs, bg, meta, pers)


# speedup vs baseline: 1.0027x; 1.0027x over previous
import jax
import jax.numpy as jnp
from jax import lax
from jax.experimental import pallas as pl
from jax.experimental.pallas import tpu as pltpu
from jax.experimental.pallas import tpu_sc as plsc

B = 16384
L = 20
R = 8
PAD = 100000
TBL = PAD + 1

NC = 2
NS = 16
NW = NC * NS
RW = B // NW
NE = RW * L
CHUNK = 128
NCHUNK = NE // CHUNK
QPL = RW // CHUNK


def _body(m_hbm, p_hbm, r_hbm, node_hbm, signs_hbm,
          bg_hbm, meta_hbm, pers_hbm, out_hbm,
          m_v, p_v, r_v, node_v, signs_v,
          base_bg, base_meta, base_pers,
          idx_bg, idx_meta, idx_pers,
          val_bg, val_meta, val_pers,
          lp_v, out_v, sem_bg, sem_meta, sem_pers):
    wid = lax.axis_index("s") * NC + lax.axis_index("c")
    base_row = wid * RW

    pltpu.sync_copy(m_hbm.at[pl.ds(base_row, RW)], m_v)
    pltpu.sync_copy(p_hbm.at[pl.ds(base_row, RW)], p_v)
    pltpu.sync_copy(r_hbm.at[pl.ds(base_row, RW)], r_v)
    pltpu.sync_copy(node_hbm.at[:, pl.ds(base_row, RW)], node_v)
    pltpu.sync_copy(signs_hbm.at[:, pl.ds(base_row, RW)], signs_v)

    def bases(c, carry):
        sl = pl.ds(c * 16, 16)
        t = r_v[sl] * TBL
        base_bg[sl] = t
        base_meta[sl] = m_v[sl] * (R * TBL) + t
        base_pers[sl] = p_v[sl] * (R * TBL) + t
        return carry

    lax.fori_loop(0, RW // 16, bases, 0)

    def build_l(ll, carry):
        def build_q(cq, c2):
            j = ll * QPL + cq
            for cc in range(CHUNK // 16):
                b0 = cq * CHUNK + cc * 16
                sl16 = pl.ds(b0, 16)
                slv = pl.ds(j * CHUNK + cc * 16, 16)
                nd = node_v[ll, sl16]
                idx_bg[slv] = base_bg[sl16] + nd
                idx_meta[slv] = base_meta[sl16] + nd
                idx_pers[slv] = base_pers[sl16] + nd
            return c2
        lax.fori_loop(0, QPL, build_q, 0)
        return carry

    lax.fori_loop(0, L, build_l, 0)

    pltpu.async_copy(bg_hbm.at[idx_bg], val_bg, sem_bg)
    pltpu.async_copy(meta_hbm.at[idx_meta], val_meta, sem_meta)
    pltpu.async_copy(pers_hbm.at[idx_pers], val_pers, sem_pers)
    pltpu.make_async_copy(bg_hbm.at[idx_bg], val_bg, sem_bg).wait()
    pltpu.make_async_copy(meta_hbm.at[idx_meta], val_meta, sem_meta).wait()
    pltpu.make_async_copy(pers_hbm.at[idx_pers], val_pers, sem_pers).wait()

    def logsig_l(ll, carry):
        def logsig_q(cq, c2):
            j = ll * QPL + cq
            for cc in range(CHUNK // 16):
                b0 = cq * CHUNK + cc * 16
                sl16 = pl.ds(b0, 16)
                slv = pl.ds(j * CHUNK + cc * 16, 16)
                logits = val_bg[slv] + val_meta[slv] + val_pers[slv]
                x = signs_v[ll, sl16] * logits
                u = jnp.exp(-jnp.abs(x))
                s = u / (2.0 + u)
                s2 = s * s
                l1p = 2.0 * s * (1.0 + s2 * (1.0 / 3.0 + s2 * (0.2 + s2 * (1.0 / 7.0 + s2 * (1.0 / 9.0)))))
                lp = jnp.minimum(x, 0.0) - l1p
                nd = node_v[ll, sl16]
                lp_v[ll, sl16] = jnp.where(nd == PAD, 0.0, lp)
            return c2
        lax.fori_loop(0, QPL, logsig_q, 0)
        return carry

    lax.fori_loop(0, L, logsig_l, 0)

    def rowsum(c, carry):
        sl = pl.ds(c * 16, 16)
        acc = lp_v[0, sl]
        for ll in range(1, L):
            acc = acc + lp_v[ll, sl]
        out_v[sl] = acc
        return carry

    lax.fori_loop(0, RW // 16, rowsum, 0)

    pltpu.sync_copy(out_v, out_hbm.at[pl.ds(base_row, RW)])


@jax.jit
def _sage_sc(m, p, r, node, signs, bg, meta, pers):
    mesh = plsc.VectorSubcoreMesh(
        core_axis_name="c", subcore_axis_name="s",
        num_cores=NC, num_subcores=NS)
    f = pl.kernel(
        _body,
        out_type=jax.ShapeDtypeStruct((B,), jnp.float32),
        mesh=mesh,
        compiler_params=pltpu.CompilerParams(needs_layout_passes=False),
        scratch_types=[
            pltpu.VMEM((RW,), jnp.int32),
            pltpu.VMEM((RW,), jnp.int32),
            pltpu.VMEM((RW,), jnp.int32),
            pltpu.VMEM((L, RW), jnp.int32),
            pltpu.VMEM((L, RW), jnp.float32),
            pltpu.VMEM((RW,), jnp.int32),
            pltpu.VMEM((RW,), jnp.int32),
            pltpu.VMEM((RW,), jnp.int32),
            pltpu.VMEM((NE,), jnp.int32),
            pltpu.VMEM((NE,), jnp.int32),
            pltpu.VMEM((NE,), jnp.int32),
            pltpu.VMEM((NE,), jnp.float32),
            pltpu.VMEM((NE,), jnp.float32),
            pltpu.VMEM((NE,), jnp.float32),
            pltpu.VMEM((L, RW), jnp.float32),
            pltpu.VMEM((RW,), jnp.float32),
            pltpu.SemaphoreType.DMA,
            pltpu.SemaphoreType.DMA,
            pltpu.SemaphoreType.DMA,
        ],
    )
    return f(m, p, r, node, signs, bg, meta, pers)


def kernel(m_idx, p_idx, r_idx, node_paths, node_signs,
           eta_bg, eta_meta, eta_pers):
    m = m_idx.astype(jnp.int32)
    p = p_idx.astype(jnp.int32)
    r = r_idx.astype(jnp.int32)
    node = node_paths.astype(jnp.int32).T
    signs = node_signs.T
    bg = eta_bg.reshape(-1)
    meta = eta_meta.reshape(-1)
    pers = eta_pers.reshape(-1)
    return _sage_sc(m, p, r, node, signs, bg, meta, pers)


# speedup vs baseline: 10.7084x; 10.6795x over previous
import functools

import jax
import jax.numpy as jnp
from jax import lax
from jax.experimental import pallas as pl
from jax.experimental.pallas import tpu as pltpu
from jax.experimental.pallas import tpu_sc as plsc

B = 16384
L = 20
M = 16
P = 64
R = 8
PAD = 100000
TBL = PAD + 1

LANE = 128
SUB = 8
CB = -(-TBL // LANE)
CW = CB * LANE
ROWBLK = CW * SUB

NC = 2
NS = 16
NW = NC * NS
RW = B // NW
NE = RW * L
CHUNK = 128
QPL = RW // CHUNK


def _relayout_body(in_ref, out_ref):
    x = in_ref[...]
    out_ref[...] = x.reshape(SUB, CB, LANE).transpose(1, 0, 2).reshape(ROWBLK)


def _tc_flatten(tbl2):
    nrb = tbl2.shape[0] // SUB
    return pl.pallas_call(
        _relayout_body,
        grid=(nrb,),
        in_specs=[pl.BlockSpec((SUB, CW), lambda i: (i, 0))],
        out_specs=pl.BlockSpec((ROWBLK,), lambda i: (i,)),
        out_shape=jax.ShapeDtypeStruct((nrb * ROWBLK,), jnp.float32),
    )(tbl2)


def _body(m_hbm, p_hbm, r_hbm, node_hbm, signs_hbm,
          bg_hbm, meta_hbm, pers_hbm, out_hbm,
          m_v, p_v, r_v, node_v, signs_v,
          base_bg, base_meta, base_pers,
          idx_bg, idx_meta, idx_pers,
          val_bg, val_meta, val_pers,
          lp_v, out_v, sem_bg, sem_meta, sem_pers):
    wid = lax.axis_index("s") * NC + lax.axis_index("c")
    base_row = wid * RW

    pltpu.sync_copy(m_hbm.at[pl.ds(base_row, RW)], m_v)
    pltpu.sync_copy(p_hbm.at[pl.ds(base_row, RW)], p_v)
    pltpu.sync_copy(r_hbm.at[pl.ds(base_row, RW)], r_v)
    pltpu.sync_copy(node_hbm.at[:, pl.ds(base_row, RW)], node_v)
    pltpu.sync_copy(signs_hbm.at[:, pl.ds(base_row, RW)], signs_v)

    def bases(c, carry):
        sl = pl.ds(c * 16, 16)
        t = r_v[sl] * LANE
        base_bg[sl] = t
        base_meta[sl] = m_v[sl] * ROWBLK + t
        base_pers[sl] = p_v[sl] * ROWBLK + t
        return carry

    lax.fori_loop(0, RW // 16, bases, 0)

    def build_l(ll, carry):
        def build_q(cq, c2):
            j = ll * QPL + cq
            for cc in range(CHUNK // 16):
                b0 = cq * CHUNK + cc * 16
                sl16 = pl.ds(b0, 16)
                slv = pl.ds(j * CHUNK + cc * 16, 16)
                nd = node_v[ll, sl16]
                off = nd + (nd >> 7) * (SUB * LANE - LANE)
                idx_bg[slv] = base_bg[sl16] + off
                idx_meta[slv] = base_meta[sl16] + off
                idx_pers[slv] = base_pers[sl16] + off
            return c2
        lax.fori_loop(0, QPL, build_q, 0)
        return carry

    lax.fori_loop(0, L, build_l, 0)

    pltpu.async_copy(bg_hbm.at[idx_bg], val_bg, sem_bg)
    pltpu.async_copy(meta_hbm.at[idx_meta], val_meta, sem_meta)
    pltpu.async_copy(pers_hbm.at[idx_pers], val_pers, sem_pers)
    pltpu.make_async_copy(bg_hbm.at[idx_bg], val_bg, sem_bg).wait()
    pltpu.make_async_copy(meta_hbm.at[idx_meta], val_meta, sem_meta).wait()
    pltpu.make_async_copy(pers_hbm.at[idx_pers], val_pers, sem_pers).wait()

    def logsig_l(ll, carry):
        def logsig_q(cq, c2):
            j = ll * QPL + cq
            for cc in range(CHUNK // 16):
                b0 = cq * CHUNK + cc * 16
                sl16 = pl.ds(b0, 16)
                slv = pl.ds(j * CHUNK + cc * 16, 16)
                logits = val_bg[slv] + val_meta[slv] + val_pers[slv]
                x = signs_v[ll, sl16] * logits
                u = jnp.exp(-jnp.abs(x))
                s = u / (2.0 + u)
                s2 = s * s
                l1p = 2.0 * s * (1.0 + s2 * (1.0 / 3.0 + s2 * (0.2 + s2 * (1.0 / 7.0 + s2 * (1.0 / 9.0)))))
                lp = jnp.minimum(x, 0.0) - l1p
                nd = node_v[ll, sl16]
                lp_v[ll, sl16] = jnp.where(nd == PAD, 0.0, lp)
            return c2
        lax.fori_loop(0, QPL, logsig_q, 0)
        return carry

    lax.fori_loop(0, L, logsig_l, 0)

    def rowsum(c, carry):
        sl = pl.ds(c * 16, 16)
        acc = lp_v[0, sl]
        for ll in range(1, L):
            acc = acc + lp_v[ll, sl]
        out_v[sl] = acc
        return carry

    lax.fori_loop(0, RW // 16, rowsum, 0)

    pltpu.sync_copy(out_v, out_hbm.at[pl.ds(base_row, RW)])


@jax.jit
def _sage_sc(m, p, r, node, signs, bg, meta, pers):
    mesh = plsc.VectorSubcoreMesh(
        core_axis_name="c", subcore_axis_name="s",
        num_cores=NC, num_subcores=NS)
    f = pl.kernel(
        _body,
        out_type=jax.ShapeDtypeStruct((B,), jnp.float32),
        mesh=mesh,
        compiler_params=pltpu.CompilerParams(needs_layout_passes=False),
        scratch_types=[
            pltpu.VMEM((RW,), jnp.int32),
            pltpu.VMEM((RW,), jnp.int32),
            pltpu.VMEM((RW,), jnp.int32),
            pltpu.VMEM((L, RW), jnp.int32),
            pltpu.VMEM((L, RW), jnp.float32),
            pltpu.VMEM((RW,), jnp.int32),
            pltpu.VMEM((RW,), jnp.int32),
            pltpu.VMEM((RW,), jnp.int32),
            pltpu.VMEM((NE,), jnp.int32),
            pltpu.VMEM((NE,), jnp.int32),
            pltpu.VMEM((NE,), jnp.int32),
            pltpu.VMEM((NE,), jnp.float32),
            pltpu.VMEM((NE,), jnp.float32),
            pltpu.VMEM((NE,), jnp.float32),
            pltpu.VMEM((L, RW), jnp.float32),
            pltpu.VMEM((RW,), jnp.float32),
            pltpu.SemaphoreType.DMA,
            pltpu.SemaphoreType.DMA,
            pltpu.SemaphoreType.DMA,
        ],
    )
    return f(m, p, r, node, signs, bg, meta, pers)


def kernel(m_idx, p_idx, r_idx, node_paths, node_signs,
           eta_bg, eta_meta, eta_pers):
    m = m_idx.astype(jnp.int32)
    p = p_idx.astype(jnp.int32)
    r = r_idx.astype(jnp.int32)
    node = node_paths.astype(jnp.int32).T
    signs = node_signs.T
    bg = _tc_flatten(eta_bg)
    meta = _tc_flatten(eta_meta.reshape(M * R, TBL))
    pers = _tc_flatten(eta_pers.reshape(P * R, TBL))
    return _sage_sc(m, p, r, node, signs, bg, meta, pers)


# speedup vs baseline: 11.2648x; 1.0520x over previous
import jax
import jax.numpy as jnp
from jax import lax
from jax.experimental import pallas as pl
from jax.experimental.pallas import tpu as pltpu
from jax.experimental.pallas import tpu_sc as plsc

B = 16384
L = 20
M = 16
P = 64
R = 8
PAD = 100000
TBL = PAD + 1

LANE = 128
SUB = 8
CB = -(-TBL // LANE)
CW = CB * LANE
ROWBLK = CW * SUB

NC = 2
NS = 16
NW = NC * NS
RW = B // NW
NE = RW * L
CHUNK = 128
QPL = RW // CHUNK


def _relayout_body(in_ref, out_ref):
    x = in_ref[...]
    out_ref[...] = x.reshape(SUB, CB, LANE).transpose(1, 0, 2).reshape(ROWBLK)


def _tc_flatten(tbl2):
    nrb = tbl2.shape[0] // SUB
    return pl.pallas_call(
        _relayout_body,
        grid=(nrb,),
        in_specs=[pl.BlockSpec((SUB, CW), lambda i: (i, 0))],
        out_specs=pl.BlockSpec((ROWBLK,), lambda i: (i,)),
        out_shape=jax.ShapeDtypeStruct((nrb * ROWBLK,), jnp.float32),
    )(tbl2)


def _body_a(m_hbm, p_hbm, r_hbm, node_hbm, bg_hbm,
            pbg_hbm, im_hbm, ip_hbm,
            m_v, p_v, r_v, node_v,
            base_bg, base_meta, base_pers,
            idx_bg, idx_meta, idx_pers, val_bg, sem_bg):
    wid = lax.axis_index("s") * NC + lax.axis_index("c")
    base_row = wid * RW
    base_el = wid * NE

    pltpu.sync_copy(m_hbm.at[pl.ds(base_row, RW)], m_v)
    pltpu.sync_copy(p_hbm.at[pl.ds(base_row, RW)], p_v)
    pltpu.sync_copy(r_hbm.at[pl.ds(base_row, RW)], r_v)
    pltpu.sync_copy(node_hbm.at[:, pl.ds(base_row, RW)], node_v)

    def bases(c, carry):
        sl = pl.ds(c * 16, 16)
        t = r_v[sl] * LANE
        base_bg[sl] = t
        base_meta[sl] = m_v[sl] * ROWBLK + t
        base_pers[sl] = p_v[sl] * ROWBLK + t
        return carry

    lax.fori_loop(0, RW // 16, bases, 0)

    def build_l(ll, carry):
        def build_q(cq, c2):
            j = ll * QPL + cq
            for cc in range(CHUNK // 16):
                b0 = cq * CHUNK + cc * 16
                sl16 = pl.ds(b0, 16)
                slv = pl.ds(j * CHUNK + cc * 16, 16)
                nd = node_v[ll, sl16]
                off = nd + (nd >> 7) * (SUB * LANE - LANE)
                idx_bg[slv] = base_bg[sl16] + off
                idx_meta[slv] = base_meta[sl16] + off
                idx_pers[slv] = base_pers[sl16] + off
            return c2
        lax.fori_loop(0, QPL, build_q, 0)
        return carry

    lax.fori_loop(0, L, build_l, 0)

    cp = pltpu.async_copy(bg_hbm.at[idx_bg], val_bg, sem_bg)
    pltpu.sync_copy(idx_meta, im_hbm.at[pl.ds(base_el, NE)])
    pltpu.sync_copy(idx_pers, ip_hbm.at[pl.ds(base_el, NE)])
    cp.wait()
    pltpu.sync_copy(val_bg, pbg_hbm.at[pl.ds(base_el, NE)])


def _body_b(node_hbm, signs_hbm, meta_hbm, pers_hbm, pbg_hbm, im_hbm, ip_hbm,
            out_hbm,
            node_v, signs_v, pbg_v,
            idx_meta, idx_pers, val_meta, val_pers,
            out_v, sem_meta, sem_pers):
    wid = lax.axis_index("s") * NC + lax.axis_index("c")
    base_row = wid * RW
    base_el = wid * NE

    pltpu.sync_copy(im_hbm.at[pl.ds(base_el, NE)], idx_meta)
    pltpu.sync_copy(ip_hbm.at[pl.ds(base_el, NE)], idx_pers)
    cm = pltpu.async_copy(meta_hbm.at[idx_meta], val_meta, sem_meta)
    cp = pltpu.async_copy(pers_hbm.at[idx_pers], val_pers, sem_pers)
    pltpu.sync_copy(node_hbm.at[:, pl.ds(base_row, RW)], node_v)
    pltpu.sync_copy(signs_hbm.at[:, pl.ds(base_row, RW)], signs_v)
    pltpu.sync_copy(pbg_hbm.at[pl.ds(base_el, NE)], pbg_v)
    cm.wait()
    cp.wait()

    def logsig_l(ll, carry):
        def logsig_q(cq, c2):
            j = ll * QPL + cq
            for cc in range(CHUNK // 16):
                b0 = cq * CHUNK + cc * 16
                sl16 = pl.ds(b0, 16)
                slv = pl.ds(j * CHUNK + cc * 16, 16)
                logits = pbg_v[slv] + val_meta[slv] + val_pers[slv]
                x = signs_v[ll, sl16] * logits
                u = jnp.exp(-jnp.abs(x))
                s = u / (2.0 + u)
                s2 = s * s
                l1p = 2.0 * s * (1.0 + s2 * (1.0 / 3.0 + s2 * (0.2 + s2 * (1.0 / 7.0 + s2 * (1.0 / 9.0)))))
                lp = jnp.minimum(x, 0.0) - l1p
                nd = node_v[ll, sl16]
                val_meta[slv] = jnp.where(nd == PAD, 0.0, lp)
            return c2
        lax.fori_loop(0, QPL, logsig_q, 0)
        return carry

    lax.fori_loop(0, L, logsig_l, 0)

    def rowsum(c, carry):
        acc = val_meta[pl.ds(c * 16, 16)]
        for ll in range(1, L):
            acc = acc + val_meta[pl.ds(ll * RW + c * 16, 16)]
        out_v[pl.ds(c * 16, 16)] = acc
        return carry

    lax.fori_loop(0, RW // 16, rowsum, 0)

    pltpu.sync_copy(out_v, out_hbm.at[pl.ds(base_row, RW)])


def _mesh():
    return plsc.VectorSubcoreMesh(
        core_axis_name="c", subcore_axis_name="s",
        num_cores=NC, num_subcores=NS)


@jax.jit
def _sage_sc(m, p, r, node, signs, bg, meta, pers):
    fa = pl.kernel(
        _body_a,
        out_type=(jax.ShapeDtypeStruct((B * L,), jnp.float32),
                  jax.ShapeDtypeStruct((B * L,), jnp.int32),
                  jax.ShapeDtypeStruct((B * L,), jnp.int32)),
        mesh=_mesh(),
        compiler_params=pltpu.CompilerParams(needs_layout_passes=False),
        scratch_types=[
            pltpu.VMEM((RW,), jnp.int32),
            pltpu.VMEM((RW,), jnp.int32),
            pltpu.VMEM((RW,), jnp.int32),
            pltpu.VMEM((L, RW), jnp.int32),
            pltpu.VMEM((RW,), jnp.int32),
            pltpu.VMEM((RW,), jnp.int32),
            pltpu.VMEM((RW,), jnp.int32),
            pltpu.VMEM((NE,), jnp.int32),
            pltpu.VMEM((NE,), jnp.int32),
            pltpu.VMEM((NE,), jnp.int32),
            pltpu.VMEM((NE,), jnp.float32),
            pltpu.SemaphoreType.DMA,
        ],
    )
    pbg, im, ip = fa(m, p, r, node, bg)
    fb = pl.kernel(
        _body_b,
        out_type=jax.ShapeDtypeStruct((B,), jnp.float32),
        mesh=_mesh(),
        compiler_params=pltpu.CompilerParams(needs_layout_passes=False),
        scratch_types=[
            pltpu.VMEM((L, RW), jnp.int32),
            pltpu.VMEM((L, RW), jnp.float32),
            pltpu.VMEM((NE,), jnp.float32),
            pltpu.VMEM((NE,), jnp.int32),
            pltpu.VMEM((NE,), jnp.int32),
            pltpu.VMEM((NE,), jnp.float32),
            pltpu.VMEM((NE,), jnp.float32),
            pltpu.VMEM((RW,), jnp.float32),
            pltpu.SemaphoreType.DMA,
            pltpu.SemaphoreType.DMA,
        ],
    )
    return fb(node, signs, meta, pers, pbg, im, ip)


def kernel(m_idx, p_idx, r_idx, node_paths, node_signs,
           eta_bg, eta_meta, eta_pers):
    m = m_idx.astype(jnp.int32)
    p = p_idx.astype(jnp.int32)
    r = r_idx.astype(jnp.int32)
    node = node_paths.astype(jnp.int32).T
    signs = node_signs.T
    bg = _tc_flatten(eta_bg)
    meta = _tc_flatten(eta_meta.reshape(M * R, TBL))
    pers = _tc_flatten(eta_pers.reshape(P * R, TBL))
    return _sage_sc(m, p, r, node, signs, bg, meta, pers)


# speedup vs baseline: 11.4787x; 1.0190x over previous
import jax
import jax.numpy as jnp
from jax import lax
from jax.experimental import pallas as pl
from jax.experimental.pallas import tpu as pltpu
from jax.experimental.pallas import tpu_sc as plsc

B = 16384
L = 20
M = 16
P = 64
R = 8
PAD = 100000
TBL = PAD + 1

LANE = 128
SUB = 8
CB = -(-TBL // LANE)
CW = CB * LANE
ROWBLK = CW * SUB

NC = 2
NS = 16
NW = NC * NS
RW = B // NW
NE = RW * L
CHUNK = 128
QPL = RW // CHUNK


def _relayout_body(in_ref, out_ref):
    x = in_ref[...]
    out_ref[...] = x.reshape(SUB, CB, LANE).transpose(1, 0, 2).reshape(ROWBLK)


def _tc_flatten(tbl2):
    nrb = tbl2.shape[0] // SUB
    return pl.pallas_call(
        _relayout_body,
        grid=(nrb,),
        in_specs=[pl.BlockSpec((SUB, CW), lambda i: (i, 0))],
        out_specs=pl.BlockSpec((ROWBLK,), lambda i: (i,)),
        out_shape=jax.ShapeDtypeStruct((nrb * ROWBLK,), jnp.float32),
    )(tbl2)


def _body_a(m_hbm, p_hbm, r_hbm, node_hbm, bg_hbm, meta_hbm,
            pbg_hbm, ip_hbm,
            m_v, p_v, r_v, node_v,
            base_bg, base_meta, base_pers,
            idx_bg, idx_meta, idx_pers, val_bg, val_meta, sem_bg, sem_meta):
    wid = lax.axis_index("s") * NC + lax.axis_index("c")
    base_row = wid * RW
    base_el = wid * NE

    pltpu.sync_copy(m_hbm.at[pl.ds(base_row, RW)], m_v)
    pltpu.sync_copy(p_hbm.at[pl.ds(base_row, RW)], p_v)
    pltpu.sync_copy(r_hbm.at[pl.ds(base_row, RW)], r_v)
    pltpu.sync_copy(node_hbm.at[:, pl.ds(base_row, RW)], node_v)

    def bases(c, carry):
        sl = pl.ds(c * 16, 16)
        t = r_v[sl] * LANE
        base_bg[sl] = t
        base_meta[sl] = m_v[sl] * ROWBLK + t
        base_pers[sl] = p_v[sl] * ROWBLK + t
        return carry

    lax.fori_loop(0, RW // 16, bases, 0)

    def build_l(ll, carry):
        def build_q(cq, c2):
            j = ll * QPL + cq
            for cc in range(CHUNK // 16):
                b0 = cq * CHUNK + cc * 16
                sl16 = pl.ds(b0, 16)
                slv = pl.ds(j * CHUNK + cc * 16, 16)
                nd = node_v[ll, sl16]
                off = nd + (nd >> 7) * (SUB * LANE - LANE)
                idx_bg[slv] = base_bg[sl16] + off
                idx_meta[slv] = base_meta[sl16] + off
                idx_pers[slv] = base_pers[sl16] + off
            return c2
        lax.fori_loop(0, QPL, build_q, 0)
        return carry

    lax.fori_loop(0, L, build_l, 0)

    cb_ = pltpu.async_copy(bg_hbm.at[idx_bg], val_bg, sem_bg)
    cm_ = pltpu.async_copy(meta_hbm.at[idx_meta], val_meta, sem_meta)
    pltpu.sync_copy(idx_pers, ip_hbm.at[pl.ds(base_el, NE)])
    cb_.wait()
    cm_.wait()

    def psum(j, carry):
        for cc in range(CHUNK // 16):
            slv = pl.ds(j * CHUNK + cc * 16, 16)
            val_bg[slv] = val_bg[slv] + val_meta[slv]
        return carry

    lax.fori_loop(0, NE // CHUNK, psum, 0)
    pltpu.sync_copy(val_bg, pbg_hbm.at[pl.ds(base_el, NE)])


def _body_b(node_hbm, signs_hbm, pers_hbm, pbg_hbm, ip_hbm,
            out_hbm,
            node_v, signs_v, pbg_v,
            idx_pers, val_pers, lp_v,
            out_v, sem_pers):
    wid = lax.axis_index("s") * NC + lax.axis_index("c")
    base_row = wid * RW
    base_el = wid * NE

    pltpu.sync_copy(ip_hbm.at[pl.ds(base_el, NE)], idx_pers)
    cp = pltpu.async_copy(pers_hbm.at[idx_pers], val_pers, sem_pers)
    pltpu.sync_copy(node_hbm.at[:, pl.ds(base_row, RW)], node_v)
    pltpu.sync_copy(signs_hbm.at[:, pl.ds(base_row, RW)], signs_v)
    pltpu.sync_copy(pbg_hbm.at[pl.ds(base_el, NE)], pbg_v)
    cp.wait()

    def logsig_l(ll, carry):
        def logsig_q(cq, c2):
            j = ll * QPL + cq
            for cc in range(CHUNK // 16):
                b0 = cq * CHUNK + cc * 16
                sl16 = pl.ds(b0, 16)
                slv = pl.ds(j * CHUNK + cc * 16, 16)
                logits = pbg_v[slv] + val_pers[slv]
                x = signs_v[ll, sl16] * logits
                u = jnp.exp(-jnp.abs(x))
                s = u / (2.0 + u)
                s2 = s * s
                l1p = 2.0 * s * (1.0 + s2 * (1.0 / 3.0 + s2 * (0.2 + s2 * (1.0 / 7.0 + s2 * (1.0 / 9.0)))))
                lp = jnp.minimum(x, 0.0) - l1p
                nd = node_v[ll, sl16]
                lp_v[slv] = jnp.where(nd == PAD, 0.0, lp)
            return c2
        lax.fori_loop(0, QPL, logsig_q, 0)
        return carry

    lax.fori_loop(0, L, logsig_l, 0)

    def rowsum(c, carry):
        acc = lp_v[pl.ds(c * 16, 16)]
        for ll in range(1, L):
            acc = acc + lp_v[pl.ds(ll * RW + c * 16, 16)]
        out_v[pl.ds(c * 16, 16)] = acc
        return carry

    lax.fori_loop(0, RW // 16, rowsum, 0)

    pltpu.sync_copy(out_v, out_hbm.at[pl.ds(base_row, RW)])


def _mesh():
    return plsc.VectorSubcoreMesh(
        core_axis_name="c", subcore_axis_name="s",
        num_cores=NC, num_subcores=NS)


@jax.jit
def _sage_sc(m, p, r, node, signs, bg, meta, pers):
    fa = pl.kernel(
        _body_a,
        out_type=(jax.ShapeDtypeStruct((B * L,), jnp.float32),
                  jax.ShapeDtypeStruct((B * L,), jnp.int32)),
        mesh=_mesh(),
        compiler_params=pltpu.CompilerParams(needs_layout_passes=False),
        scratch_types=[
            pltpu.VMEM((RW,), jnp.int32),
            pltpu.VMEM((RW,), jnp.int32),
            pltpu.VMEM((RW,), jnp.int32),
            pltpu.VMEM((L, RW), jnp.int32),
            pltpu.VMEM((RW,), jnp.int32),
            pltpu.VMEM((RW,), jnp.int32),
            pltpu.VMEM((RW,), jnp.int32),
            pltpu.VMEM((NE,), jnp.int32),
            pltpu.VMEM((NE,), jnp.int32),
            pltpu.VMEM((NE,), jnp.int32),
            pltpu.VMEM((NE,), jnp.float32),
            pltpu.VMEM((NE,), jnp.float32),
            pltpu.SemaphoreType.DMA,
            pltpu.SemaphoreType.DMA,
        ],
    )
    pbg, ip = fa(m, p, r, node, bg, meta)
    fb = pl.kernel(
        _body_b,
        out_type=jax.ShapeDtypeStruct((B,), jnp.float32),
        mesh=_mesh(),
        compiler_params=pltpu.CompilerParams(needs_layout_passes=False),
        scratch_types=[
            pltpu.VMEM((L, RW), jnp.int32),
            pltpu.VMEM((L, RW), jnp.float32),
            pltpu.VMEM((NE,), jnp.float32),
            pltpu.VMEM((NE,), jnp.int32),
            pltpu.VMEM((NE,), jnp.float32),
            pltpu.VMEM((NE,), jnp.float32),
            pltpu.VMEM((RW,), jnp.float32),
            pltpu.SemaphoreType.DMA,
        ],
    )
    return fb(node, signs, pers, pbg, ip)


def kernel(m_idx, p_idx, r_idx, node_paths, node_signs,
           eta_bg, eta_meta, eta_pers):
    m = m_idx.astype(jnp.int32)
    p = p_idx.astype(jnp.int32)
    r = r_idx.astype(jnp.int32)
    node = node_paths.astype(jnp.int32).T
    signs = node_signs.T
    bg = _tc_flatten(eta_bg)
    meta = _tc_flatten(eta_meta.reshape(M * R, TBL))
    pers = _tc_flatten(eta_pers.reshape(P * R, TBL))
    return _sage_sc(m, p, r, node, signs, bg, meta, pers)
